# baseline (speedup 1.0000x reference)
import functools

import jax
import jax.numpy as jnp
from jax import lax
from jax.experimental import pallas as pl
from jax.experimental.pallas import tpu as pltpu
from jax.experimental.pallas import tpu_sc as plsc

B = 16384
D = 128
NC = 2
NS = 16
L = 16
NW = NC * NS
BPW = B // NW
CH = 128
NCHUNK = BPW // CH

_mesh = plsc.VectorSubcoreMesh(core_axis_name="c", subcore_axis_name="s")


@functools.partial(
    pl.kernel,
    mesh=_mesh,
    compiler_params=pltpu.CompilerParams(needs_layout_passes=False),
    out_type=jax.ShapeDtypeStruct((B,), jnp.float32),
    scratch_types=[
        pltpu.VMEM((NCHUNK, CH), jnp.int32),
        pltpu.VMEM((NCHUNK, CH), jnp.int32),
        pltpu.VMEM((CH, D), jnp.float32),
        pltpu.VMEM((CH, D), jnp.float32),
        pltpu.VMEM((BPW,), jnp.float32),
        pltpu.SemaphoreType.DMA,
    ],
)
def _mf_sc(user_hbm, item_hbm, uemb_hbm, iemb_hbm, out_hbm,
           uidx_v, iidx_v, urows_v, irows_v, outv, sem):
    wid = lax.axis_index("s") * NC + lax.axis_index("c")
    base = wid * BPW

    for c in range(NCHUNK):
        pltpu.sync_copy(user_hbm.at[pl.ds(base + c * CH, CH)], uidx_v.at[c])
        pltpu.sync_copy(item_hbm.at[pl.ds(base + c * CH, CH)], iidx_v.at[c])

    lane = lax.iota(jnp.int32, L)

    for c in range(NCHUNK):
        pltpu.async_copy(uemb_hbm.at[uidx_v.at[c]], urows_v, sem).wait()
        pltpu.async_copy(iemb_hbm.at[iidx_v.at[c]], irows_v, sem).wait()

        def group_body(g, carry, c=c):
            rid = g * L + lane
            acc = jnp.zeros((L,), jnp.float32)
            for d in range(D):
                col = jnp.full((L,), d, jnp.int32)
                uu = plsc.load_gather(urows_v, [rid, col])
                vv = plsc.load_gather(irows_v, [rid, col])
                acc = acc + uu * vv
            outv[pl.ds(c * CH + g * L, L)] = acc
            return carry

        lax.fori_loop(0, CH // L, group_body, 0)

    pltpu.sync_copy(outv, out_hbm.at[pl.ds(base, BPW)])


def kernel(user, item, users_emb, items_emb):
    return _mf_sc(user, item, users_emb, items_emb)


# speedup vs baseline: 2.2222x; 2.2222x over previous
import functools

import jax
import jax.numpy as jnp
from jax import lax
from jax.experimental import pallas as pl
from jax.experimental.pallas import tpu as pltpu
from jax.experimental.pallas import tpu_sc as plsc

B = 16384
D = 128
NC = 2
NS = 16
L = 16
NW = NC * NS
BPW = B // NW
CH = 128
NCHUNK = BPW // CH

_mesh = plsc.VectorSubcoreMesh(core_axis_name="c", subcore_axis_name="s")


@functools.partial(
    pl.kernel,
    mesh=_mesh,
    compiler_params=pltpu.CompilerParams(needs_layout_passes=False),
    out_type=jax.ShapeDtypeStruct((B,), jnp.float32),
    scratch_types=[
        pltpu.VMEM((NCHUNK, CH), jnp.int32),
        pltpu.VMEM((NCHUNK, CH), jnp.int32),
        pltpu.VMEM((2, CH, D), jnp.float32),
        pltpu.VMEM((2, CH, D), jnp.float32),
        pltpu.VMEM((L, 2 * L), jnp.float32),
        pltpu.VMEM((BPW,), jnp.float32),
        pltpu.SemaphoreType.DMA,
        pltpu.SemaphoreType.DMA,
    ],
)
def _mf_sc(user_hbm, item_hbm, uemb_hbm, iemb_hbm, out_hbm,
           uidx_v, iidx_v, urows_v, irows_v, pbuf, outv, sem0, sem1):
    wid = lax.axis_index("s") * NC + lax.axis_index("c")
    base = wid * BPW

    for c in range(NCHUNK):
        pltpu.sync_copy(user_hbm.at[pl.ds(base + c * CH, CH)], uidx_v.at[c])
        pltpu.sync_copy(item_hbm.at[pl.ds(base + c * CH, CH)], iidx_v.at[c])

    sems = (sem0, sem1)

    def start(c):
        b = c % 2
        cu = pltpu.async_copy(uemb_hbm.at[uidx_v.at[c]], urows_v.at[b], sems[b])
        cv = pltpu.async_copy(iemb_hbm.at[iidx_v.at[c]], irows_v.at[b], sems[b])
        return (cu, cv)

    lane = lax.iota(jnp.int32, L)
    inflight = start(0)

    for c in range(NCHUNK):
        b = c % 2
        if c + 1 < NCHUNK:
            nxt = start(c + 1)
        inflight[0].wait()
        inflight[1].wait()
        if c + 1 < NCHUNK:
            inflight = nxt

        def group_body(g, carry, b=b, c=c):
            vec = jnp.zeros((L,), jnp.float32)
            for k in range(L):
                r = g * L + k
                part = jnp.zeros((L,), jnp.float32)
                for j in range(D // L):
                    uu = urows_v[b, r, pl.ds(j * L, L)]
                    vv = irows_v[b, r, pl.ds(j * L, L)]
                    part = part + uu * vv
                for sh in (8, 4, 2, 1):
                    pbuf[k, pl.ds(0, L)] = part
                    pbuf[k, pl.ds(L, L)] = part
                    part = part + pbuf[k, pl.ds(sh, L)]
                vec = jnp.where(lane == k, part, vec)
            outv[pl.ds(c * CH + g * L, L)] = vec
            return carry

        lax.fori_loop(0, CH // L, group_body, 0)

    pltpu.sync_copy(outv, out_hbm.at[pl.ds(base, BPW)])


def kernel(user, item, users_emb, items_emb):
    return _mf_sc(user, item, users_emb, items_emb)


# speedup vs baseline: 2.7874x; 1.2544x over previous
import functools

import jax
import jax.numpy as jnp
from jax import lax
from jax.experimental import pallas as pl
from jax.experimental.pallas import tpu as pltpu
from jax.experimental.pallas import tpu_sc as plsc

B = 16384
D = 128
NC = 2
NS = 16
L = 16
NW = NC * NS
BPW = B // NW
CH = 128
NCHUNK = BPW // CH

_mesh = plsc.VectorSubcoreMesh(core_axis_name="c", subcore_axis_name="s")


@functools.partial(
    pl.kernel,
    mesh=_mesh,
    compiler_params=pltpu.CompilerParams(needs_layout_passes=False),
    out_type=jax.ShapeDtypeStruct((B,), jnp.float32),
    scratch_types=[
        pltpu.VMEM((NCHUNK, CH), jnp.int32),
        pltpu.VMEM((NCHUNK, CH), jnp.int32),
        pltpu.VMEM((2, CH, D), jnp.float32),
        pltpu.VMEM((2, CH, D), jnp.float32),
        pltpu.VMEM((L * 17,), jnp.float32),
        pltpu.VMEM((BPW,), jnp.float32),
        pltpu.SemaphoreType.DMA,
        pltpu.SemaphoreType.DMA,
    ],
)
def _mf_sc(user_hbm, item_hbm, uemb_hbm, iemb_hbm, out_hbm,
           uidx_v, iidx_v, urows_v, irows_v, pbuf, outv, sem0, sem1):
    wid = lax.axis_index("s") * NC + lax.axis_index("c")
    base = wid * BPW

    for c in range(NCHUNK):
        pltpu.sync_copy(user_hbm.at[pl.ds(base + c * CH, CH)], uidx_v.at[c])
        pltpu.sync_copy(item_hbm.at[pl.ds(base + c * CH, CH)], iidx_v.at[c])

    sems = (sem0, sem1)

    def start(c):
        b = c % 2
        cu = pltpu.async_copy(uemb_hbm.at[uidx_v.at[c]], urows_v.at[b], sems[b])
        cv = pltpu.async_copy(iemb_hbm.at[iidx_v.at[c]], irows_v.at[b], sems[b])
        return (cu, cv)

    lane17 = lax.iota(jnp.int32, L) * 17
    inflight = start(0)

    for c in range(NCHUNK):
        b = c % 2
        if c + 1 < NCHUNK:
            nxt = start(c + 1)
        inflight[0].wait()
        inflight[1].wait()
        if c + 1 < NCHUNK:
            inflight = nxt

        def group_body(g, carry, b=b, c=c):
            for k in range(L):
                r = g * L + k
                part = jnp.zeros((L,), jnp.float32)
                for j in range(D // L):
                    uu = urows_v[b, r, pl.ds(j * L, L)]
                    vv = irows_v[b, r, pl.ds(j * L, L)]
                    part = part + uu * vv
                plsc.store_scatter(pbuf, [lane17 + k], part)
            acc = jnp.zeros((L,), jnp.float32)
            for j in range(L):
                acc = acc + pbuf[pl.ds(j * 17, L)]
            outv[pl.ds(c * CH + g * L, L)] = acc
            return carry

        lax.fori_loop(0, CH // L, group_body, 0)

    pltpu.sync_copy(outv, out_hbm.at[pl.ds(base, BPW)])


def kernel(user, item, users_emb, items_emb):
    return _mf_sc(user, item, users_emb, items_emb)


# speedup vs baseline: 3.2093x; 1.1514x over previous
import functools

import jax
import jax.numpy as jnp
from jax import lax
from jax.experimental import pallas as pl
from jax.experimental.pallas import tpu as pltpu
from jax.experimental.pallas import tpu_sc as plsc

B = 16384
D = 128
NC = 2
NS = 16
L = 16
NW = NC * NS
BPW = B // NW
CH = 128
NCHUNK = BPW // CH
NG = CH // L

_mesh = plsc.VectorSubcoreMesh(core_axis_name="c", subcore_axis_name="s")


@functools.partial(
    pl.kernel,
    mesh=_mesh,
    compiler_params=pltpu.CompilerParams(needs_layout_passes=False),
    out_type=jax.ShapeDtypeStruct((B,), jnp.float32),
    scratch_types=[
        pltpu.VMEM((BPW,), jnp.int32),
        pltpu.VMEM((BPW,), jnp.int32),
        pltpu.VMEM((2, CH, D), jnp.float32),
        pltpu.VMEM((2, CH, D), jnp.float32),
        pltpu.VMEM((L * 17,), jnp.float32),
        pltpu.VMEM((BPW,), jnp.float32),
        pltpu.SemaphoreType.DMA,
        pltpu.SemaphoreType.DMA,
        pltpu.SemaphoreType.DMA,
    ],
)
def _mf_sc(user_hbm, item_hbm, uemb_hbm, iemb_hbm, out_hbm,
           uidx_v, iidx_v, urows_v, irows_v, pbuf, outv, sem0, sem1, semi):
    wid = lax.axis_index("s") * NC + lax.axis_index("c")
    base = wid * BPW

    ci = pltpu.async_copy(user_hbm.at[pl.ds(base, BPW)], uidx_v, semi)
    cj = pltpu.async_copy(item_hbm.at[pl.ds(base, BPW)], iidx_v, semi)
    ci.wait()
    cj.wait()

    lane17 = lax.iota(jnp.int32, L) * 17
    sems = (sem0, sem1)

    def start(c, b):
        cu = pltpu.async_copy(
            uemb_hbm.at[uidx_v.at[pl.ds(c * CH, CH)]], urows_v.at[b], sems[b])
        cv = pltpu.async_copy(
            iemb_hbm.at[iidx_v.at[pl.ds(c * CH, CH)]], irows_v.at[b], sems[b])
        return (cu, cv)

    start(0, 0)
    start(1, 1)

    def compute(c, b):
        pltpu.make_async_copy(
            uemb_hbm.at[uidx_v.at[pl.ds(c * CH, CH)]], urows_v.at[b], sems[b]).wait()
        pltpu.make_async_copy(
            iemb_hbm.at[iidx_v.at[pl.ds(c * CH, CH)]], irows_v.at[b], sems[b]).wait()

        def group_body(g, carry):
            for k in range(L):
                r = g * L + k
                part = jnp.zeros((L,), jnp.float32)
                for j in range(D // L):
                    uu = urows_v[b, r, pl.ds(j * L, L)]
                    vv = irows_v[b, r, pl.ds(j * L, L)]
                    part = part + uu * vv
                plsc.store_scatter(pbuf, [lane17 + k], part)
            acc = jnp.zeros((L,), jnp.float32)
            for j in range(L):
                acc = acc + pbuf[pl.ds(j * 17, L)]
            outv[pl.ds(c * CH + g * L, L)] = acc
            return carry

        lax.fori_loop(0, NG, group_body, 0)

    def pair_body(i, carry):
        c0 = 2 * i
        compute(c0, 0)

        @pl.when(i == 0)
        def _():
            start(c0 + 2, 0)

        compute(c0 + 1, 1)

        @pl.when(i == 0)
        def _():
            start(c0 + 3, 1)

        return carry

    lax.fori_loop(0, NCHUNK // 2, pair_body, 0)

    pltpu.sync_copy(outv, out_hbm.at[pl.ds(base, BPW)])


def kernel(user, item, users_emb, items_emb):
    return _mf_sc(user, item, users_emb, items_emb)


# speedup vs baseline: 3.3554x; 1.0455x over previous
import functools

import jax
import jax.numpy as jnp
from jax import lax
from jax.experimental import pallas as pl
from jax.experimental.pallas import tpu as pltpu
from jax.experimental.pallas import tpu_sc as plsc

B = 16384
D = 128
NC = 2
NS = 16
L = 16
NW = NC * NS
BPW = B // NW
CH = 128
NCHUNK = BPW // CH
NG = CH // L

_mesh = plsc.VectorSubcoreMesh(core_axis_name="c", subcore_axis_name="s")


@functools.partial(
    pl.kernel,
    mesh=_mesh,
    compiler_params=pltpu.CompilerParams(needs_layout_passes=False),
    out_type=jax.ShapeDtypeStruct((B,), jnp.float32),
    scratch_types=[
        pltpu.VMEM((BPW,), jnp.int32),
        pltpu.VMEM((BPW,), jnp.int32),
        pltpu.VMEM((2, CH, D), jnp.float32),
        pltpu.VMEM((2, CH, D), jnp.float32),
        pltpu.VMEM((L * 17,), jnp.float32),
        pltpu.VMEM((BPW,), jnp.float32),
        pltpu.SemaphoreType.DMA((2,)),
        pltpu.SemaphoreType.DMA,
    ],
)
def _mf_sc(user_hbm, item_hbm, uemb_hbm, iemb_hbm, out_hbm,
           uidx_v, iidx_v, urows_v, irows_v, pbuf, outv, sems, semi):
    wid = lax.axis_index("s") * NC + lax.axis_index("c")
    base = wid * BPW

    ci = pltpu.async_copy(user_hbm.at[pl.ds(base, BPW)], uidx_v, semi)
    cj = pltpu.async_copy(item_hbm.at[pl.ds(base, BPW)], iidx_v, semi)
    ci.wait()
    cj.wait()

    lane17 = lax.iota(jnp.int32, L) * 17

    def start(c, b):
        pltpu.async_copy(
            uemb_hbm.at[uidx_v.at[pl.ds(c * CH, CH)]], urows_v.at[b], sems.at[b])
        pltpu.async_copy(
            iemb_hbm.at[iidx_v.at[pl.ds(c * CH, CH)]], irows_v.at[b], sems.at[b])

    start(0, 0)
    start(1, 1)

    def chunk_body(c, carry):
        b = lax.rem(c, 2)
        pltpu.make_async_copy(
            uemb_hbm.at[uidx_v.at[pl.ds(c * CH, CH)]], urows_v.at[b], sems.at[b]).wait()
        pltpu.make_async_copy(
            iemb_hbm.at[iidx_v.at[pl.ds(c * CH, CH)]], irows_v.at[b], sems.at[b]).wait()

        def group_body(g, carry2):
            def row_body(k, carry3):
                r = g * L + k
                part = jnp.zeros((L,), jnp.float32)
                for j in range(D // L):
                    uu = urows_v[b, r, pl.ds(j * L, L)]
                    vv = irows_v[b, r, pl.ds(j * L, L)]
                    part = part + uu * vv
                plsc.store_scatter(pbuf, [lane17 + k], part)
                return carry3

            lax.fori_loop(0, L, row_body, 0)
            acc = jnp.zeros((L,), jnp.float32)
            for j in range(L):
                acc = acc + pbuf[pl.ds(j * 17, L)]
            outv[pl.ds(c * CH + g * L, L)] = acc
            return carry2

        lax.fori_loop(0, NG, group_body, 0)

        @pl.when(c + 2 < NCHUNK)
        def _():
            start(c + 2, b)

        return carry

    lax.fori_loop(0, NCHUNK, chunk_body, 0)

    pltpu.sync_copy(outv, out_hbm.at[pl.ds(base, BPW)])


def kernel(user, item, users_emb, items_emb):
    return _mf_sc(user, item, users_emb, items_emb)
